# Initial kernel scaffold; baseline (speedup 1.0000x reference)
import jax
import jax.numpy as jnp
from jax.experimental import pallas as pl


def kernel(species, coordinates, neighbor_idxs, shift_values, cutoff):
    raise NotImplementedError("write your pallas kernel here")


# speedup vs baseline: 2.1609x; 2.1609x over previous
import functools

import jax
import jax.numpy as jnp
from jax import lax
from jax.experimental import pallas as pl
from jax.experimental.pallas import tpu as pltpu
from jax.experimental.pallas import tpu_sc as plsc

NA = 100000
NP = 3200000
NC = 2
NS = 16
NW = NC * NS
PPW = NP // NW
BL = 125
NB = 16
C = NB * BL
NCH = PPW // C
L = 16


def _dist_from_sumsq(ss):
    xi = lax.bitcast_convert_type(ss, jnp.int32)
    yi = jnp.int32(0x5F3759DF) - (xi >> 1)
    y = lax.bitcast_convert_type(yi, jnp.float32)
    for _ in range(3):
        y = y * (1.5 - 0.5 * ss * y * y)
    return jnp.where(ss > 0.0, ss * y, 0.0)


def _make_kernel():
    mesh = plsc.VectorSubcoreMesh(core_axis_name="c", subcore_axis_name="s")

    @functools.partial(
        pl.kernel,
        mesh=mesh,
        compiler_params=pltpu.CompilerParams(
            use_tc_tiling_on_sc=False, needs_layout_passes=False),
        out_type=[
            jax.ShapeDtypeStruct((NP,), jnp.float32),
            jax.ShapeDtypeStruct((3 * NP,), jnp.float32),
        ],
        scratch_types=[
            pltpu.VMEM_SHARED((NA, 8), jnp.float32),
            pltpu.VMEM((NB, BL), jnp.int32),
            pltpu.VMEM((NB, BL), jnp.int32),
            pltpu.VMEM((C, 8), jnp.float32),
            pltpu.VMEM((C, 8), jnp.float32),
            pltpu.VMEM((3 * C,), jnp.float32),
            pltpu.VMEM((3 * C,), jnp.float32),
            pltpu.VMEM((C,), jnp.float32),
            pltpu.SemaphoreType.DMA,
        ],
    )
    def nbr_kernel(coords_hbm, idx2d_hbm, shift_hbm, dist_hbm, diff_hbm,
                   tab_sh, idx0_v, idx1_v, rows0_v, rows1_v,
                   shift_v, diffp_v, dist_v, sem):
        cid = lax.axis_index("c")
        sid = lax.axis_index("s")
        wid = cid * NS + sid

        @pl.when(sid == 0)
        def _stage_table():
            pltpu.sync_copy(coords_hbm, tab_sh)

        plsc.subcore_barrier()

        def chunk_body(i, carry):
            base = wid * PPW + i * C
            brow = wid * (PPW // BL) + i * NB
            pltpu.sync_copy(idx2d_hbm.at[pl.ds(brow, NB)], idx0_v)
            pltpu.sync_copy(idx2d_hbm.at[pl.ds(NP // BL + brow, NB)], idx1_v)
            pltpu.sync_copy(shift_hbm.at[pl.ds(3 * base, 3 * C)], shift_v)
            cps = []
            for k in range(NB):
                cps.append(pltpu.async_copy(
                    tab_sh.at[idx0_v.at[k]],
                    rows0_v.at[pl.ds(k * BL, BL)], sem))
                cps.append(pltpu.async_copy(
                    tab_sh.at[idx1_v.at[k]],
                    rows1_v.at[pl.ds(k * BL, BL)], sem))
            for cp in cps:
                cp.wait()

            def row_body(u, carry2):
                lanes = lax.iota(jnp.int32, L)
                pbase = 3 * L * u
                for t in range(3):
                    o = pbase + t * L
                    p = o + lanes
                    r = lax.div(p, jnp.int32(3))
                    c = p - 3 * r
                    a0 = plsc.load_gather(rows0_v, [r, c])
                    a1 = plsc.load_gather(rows1_v, [r, c])
                    diffp_v[pl.ds(o, L)] = a0 - a1 + shift_v[pl.ds(o, L)]
                rb = L * u
                i3 = 3 * rb + 3 * lanes
                xs = plsc.load_gather(diffp_v, [i3])
                ys = plsc.load_gather(diffp_v, [i3 + 1])
                zs = plsc.load_gather(diffp_v, [i3 + 2])
                ss = xs * xs + ys * ys + zs * zs
                dist_v[pl.ds(rb, L)] = _dist_from_sumsq(ss)
                return carry2

            lax.fori_loop(0, C // L, row_body, 0)

            pltpu.sync_copy(dist_v, dist_hbm.at[pl.ds(base, C)])
            pltpu.sync_copy(diffp_v, diff_hbm.at[pl.ds(3 * base, 3 * C)])
            return carry

        lax.fori_loop(0, NCH, chunk_body, 0)

    return nbr_kernel


_NBR_KERNEL = _make_kernel()


@jax.jit
def _run(coords, idx2d, shift_flat):
    return _NBR_KERNEL(coords, idx2d, shift_flat)


def kernel(species, coordinates, neighbor_idxs, shift_values, cutoff):
    del species, cutoff
    coords = jnp.pad(coordinates.reshape(-1, 3), ((0, 0), (0, 5)))
    idx2d = neighbor_idxs.reshape(-1, BL)
    dist, diff = _run(coords, idx2d, shift_values.reshape(-1))
    return neighbor_idxs, dist, diff.reshape(-1, 3)


# speedup vs baseline: 38.7099x; 17.9139x over previous
import functools

import jax
import jax.numpy as jnp
from jax import lax
from jax.experimental import pallas as pl
from jax.experimental.pallas import tpu as pltpu
from jax.experimental.pallas import tpu_sc as plsc

NA = 100000
NP = 3200000
NC = 2
NS = 16
NW = NC * NS
PPW = NP // NW
BL = 80
NB = 25
C = NB * BL
NCH = PPW // C
L = 16


def _dist_from_sumsq(ss):
    xi = lax.bitcast_convert_type(ss, jnp.int32)
    yi = jnp.int32(0x5F3759DF) - (xi >> 1)
    y = lax.bitcast_convert_type(yi, jnp.float32)
    for _ in range(3):
        y = y * (1.5 - 0.5 * ss * y * y)
    return jnp.where(ss > 0.0, ss * y, 0.0)


def _make_kernel():
    mesh = plsc.VectorSubcoreMesh(core_axis_name="c", subcore_axis_name="s")

    @functools.partial(
        pl.kernel,
        mesh=mesh,
        compiler_params=pltpu.CompilerParams(
            use_tc_tiling_on_sc=False, needs_layout_passes=False),
        out_type=[
            jax.ShapeDtypeStruct((NP,), jnp.float32),
            jax.ShapeDtypeStruct((NP,), jnp.float32),
            jax.ShapeDtypeStruct((NP,), jnp.float32),
            jax.ShapeDtypeStruct((NP,), jnp.float32),
        ],
        scratch_types=[
            pltpu.VMEM_SHARED((NA, 8), jnp.float32),
            pltpu.VMEM((C,), jnp.int32),
            pltpu.VMEM((C,), jnp.int32),
            pltpu.VMEM((C, 8), jnp.float32),
            pltpu.VMEM((C, 8), jnp.float32),
            pltpu.VMEM((C,), jnp.float32),
            pltpu.VMEM((C,), jnp.float32),
            pltpu.VMEM((C,), jnp.float32),
            pltpu.VMEM((C,), jnp.float32),
            pltpu.VMEM((C,), jnp.float32),
            pltpu.VMEM((C,), jnp.float32),
            pltpu.VMEM((C,), jnp.float32),
            pltpu.SemaphoreType.DMA,
        ],
    )
    def nbr_kernel(coords_hbm, idx0_hbm, idx1_hbm, sx_hbm, sy_hbm, sz_hbm,
                   dist_hbm, dx_hbm, dy_hbm, dz_hbm,
                   tab_sh, idx0_v, idx1_v, rows0_v, rows1_v,
                   sx_v, sy_v, sz_v, dx_v, dy_v, dz_v, dist_v, sem):
        cid = lax.axis_index("c")
        sid = lax.axis_index("s")
        wid = cid * NS + sid

        @pl.when(sid == 0)
        def _stage_table():
            pltpu.sync_copy(coords_hbm, tab_sh)

        plsc.subcore_barrier()

        def chunk_body(i, carry):
            base = wid * PPW + i * C
            pltpu.sync_copy(idx0_hbm.at[pl.ds(base, C)], idx0_v)
            pltpu.sync_copy(idx1_hbm.at[pl.ds(base, C)], idx1_v)
            pltpu.sync_copy(sx_hbm.at[pl.ds(base, C)], sx_v)
            pltpu.sync_copy(sy_hbm.at[pl.ds(base, C)], sy_v)
            pltpu.sync_copy(sz_hbm.at[pl.ds(base, C)], sz_v)
            cps = []
            for k in range(NB):
                blk = pl.ds(k * BL, BL)
                cps.append(pltpu.async_copy(
                    tab_sh.at[idx0_v.at[blk]], rows0_v.at[blk], sem))
                cps.append(pltpu.async_copy(
                    tab_sh.at[idx1_v.at[blk]], rows1_v.at[blk], sem))
            for cp in cps:
                cp.wait()

            def row_body(u, carry2):
                lanes = lax.iota(jnp.int32, L)
                rb = L * u
                r = rb + lanes
                c0 = lanes * 0
                c1 = c0 + 1
                c2 = c0 + 2
                dx = (plsc.load_gather(rows0_v, [r, c0])
                      - plsc.load_gather(rows1_v, [r, c0])
                      + sx_v[pl.ds(rb, L)])
                dy = (plsc.load_gather(rows0_v, [r, c1])
                      - plsc.load_gather(rows1_v, [r, c1])
                      + sy_v[pl.ds(rb, L)])
                dz = (plsc.load_gather(rows0_v, [r, c2])
                      - plsc.load_gather(rows1_v, [r, c2])
                      + sz_v[pl.ds(rb, L)])
                dx_v[pl.ds(rb, L)] = dx
                dy_v[pl.ds(rb, L)] = dy
                dz_v[pl.ds(rb, L)] = dz
                ss = dx * dx + dy * dy + dz * dz
                dist_v[pl.ds(rb, L)] = _dist_from_sumsq(ss)
                return carry2

            lax.fori_loop(0, C // L, row_body, 0)

            pltpu.sync_copy(dist_v, dist_hbm.at[pl.ds(base, C)])
            pltpu.sync_copy(dx_v, dx_hbm.at[pl.ds(base, C)])
            pltpu.sync_copy(dy_v, dy_hbm.at[pl.ds(base, C)])
            pltpu.sync_copy(dz_v, dz_hbm.at[pl.ds(base, C)])
            return carry

        lax.fori_loop(0, NCH, chunk_body, 0)

    return nbr_kernel


_NBR_KERNEL = _make_kernel()


@jax.jit
def _run(coords, idx0, idx1, sx, sy, sz):
    return _NBR_KERNEL(coords, idx0, idx1, sx, sy, sz)


def kernel(species, coordinates, neighbor_idxs, shift_values, cutoff):
    del species, cutoff
    coords = jnp.pad(coordinates.reshape(-1, 3), ((0, 0), (0, 5)))
    dist, dx, dy, dz = _run(
        coords,
        neighbor_idxs[0], neighbor_idxs[1],
        shift_values[:, 0], shift_values[:, 1], shift_values[:, 2])
    diff = jnp.stack([dx, dy, dz], axis=1)
    return neighbor_idxs, dist, diff


# speedup vs baseline: 50.0522x; 1.2930x over previous
import functools

import jax
import jax.numpy as jnp
from jax import lax
from jax.experimental import pallas as pl
from jax.experimental.pallas import tpu as pltpu
from jax.experimental.pallas import tpu_sc as plsc

NA = 100000
NT = NA // 2
NP = 3200000
NC = 2
NS = 16
NW = NC * NS
PPW = NP // NW
BL = 80
NB = 25
C = NB * BL
NCH = PPW // C
L = 16
UNR = 5


def _dist_from_sumsq(ss):
    xi = lax.bitcast_convert_type(ss, jnp.int32)
    yi = jnp.int32(0x5F3759DF) - (xi >> 1)
    y = lax.bitcast_convert_type(yi, jnp.float32)
    for _ in range(3):
        y = y * (1.5 - 0.5 * ss * y * y)
    return jnp.where(ss > 0.0, ss * y, 0.0)


def _make_kernel():
    mesh = plsc.VectorSubcoreMesh(core_axis_name="c", subcore_axis_name="s")

    buf = lambda shape, dt: pltpu.VMEM(shape, dt)

    @functools.partial(
        pl.kernel,
        mesh=mesh,
        compiler_params=pltpu.CompilerParams(
            use_tc_tiling_on_sc=False, needs_layout_passes=False),
        out_type=[
            jax.ShapeDtypeStruct((NP,), jnp.float32),
            jax.ShapeDtypeStruct((NP,), jnp.float32),
            jax.ShapeDtypeStruct((NP,), jnp.float32),
            jax.ShapeDtypeStruct((NP,), jnp.float32),
        ],
        scratch_types=[
            pltpu.VMEM_SHARED((NT, 8), jnp.float32),
            buf((C,), jnp.int32), buf((C,), jnp.int32),
            buf((C,), jnp.int32), buf((C,), jnp.int32),
            buf((C,), jnp.float32), buf((C,), jnp.float32),
            buf((C,), jnp.float32),
            buf((C, 8), jnp.float32), buf((C, 8), jnp.float32),
            buf((C,), jnp.float32), buf((C,), jnp.float32),
            buf((C,), jnp.float32),
            buf((C,), jnp.int32), buf((C,), jnp.int32),
            buf((C,), jnp.int32), buf((C,), jnp.int32),
            buf((C,), jnp.float32), buf((C,), jnp.float32),
            buf((C,), jnp.float32),
            buf((C, 8), jnp.float32), buf((C, 8), jnp.float32),
            buf((C,), jnp.float32), buf((C,), jnp.float32),
            buf((C,), jnp.float32),
            pltpu.SemaphoreType.DMA,
            pltpu.SemaphoreType.DMA,
            pltpu.SemaphoreType.DMA,
        ],
    )
    def nbr_kernel(coords_hbm, idx0_hbm, idx1_hbm, sx_hbm, sy_hbm, sz_hbm,
                   dist_hbm, dx_hbm, dy_hbm, dz_hbm,
                   tab_sh,
                   idx0_a, idx1_a, h0_a, h1_a, sx_a, sy_a, sz_a,
                   rows0_a, rows1_a, dx_a, dy_a, dist_a,
                   idx0_b, idx1_b, h0_b, h1_b, sx_b, sy_b, sz_b,
                   rows0_b, rows1_b, dx_b, dy_b, dist_b,
                   sem_in, sem_g, sem_out):
        cid = lax.axis_index("c")
        sid = lax.axis_index("s")
        wid = cid * NS + sid

        @pl.when(sid == 0)
        def _stage_table():
            pltpu.sync_copy(coords_hbm, tab_sh)

        plsc.subcore_barrier()

        bufs_a = (idx0_a, idx1_a, h0_a, h1_a, sx_a, sy_a, sz_a,
                  rows0_a, rows1_a, dx_a, dy_a, dist_a)
        bufs_b = (idx0_b, idx1_b, h0_b, h1_b, sx_b, sy_b, sz_b,
                  rows0_b, rows1_b, dx_b, dy_b, dist_b)

        def issue_in(i, bufs):
            idx0_v, idx1_v = bufs[0], bufs[1]
            sx_v, sy_v, sz_v = bufs[4], bufs[5], bufs[6]
            base = wid * PPW + i * C
            sl = pl.ds(base, C)
            return [
                pltpu.async_copy(idx0_hbm.at[sl], idx0_v, sem_in),
                pltpu.async_copy(idx1_hbm.at[sl], idx1_v, sem_in),
                pltpu.async_copy(sx_hbm.at[sl], sx_v, sem_in),
                pltpu.async_copy(sy_hbm.at[sl], sy_v, sem_in),
                pltpu.async_copy(sz_hbm.at[sl], sz_v, sem_in),
            ]

        def halve(bufs):
            idx0_v, idx1_v, h0_v, h1_v = bufs[:4]

            def hb(u, carry2):
                o = pl.ds(L * u, L)
                h0_v[o] = idx0_v[o] >> 1
                h1_v[o] = idx1_v[o] >> 1
                return carry2

            lax.fori_loop(0, C // L, hb, 0)

        def issue_gathers(bufs):
            h0_v, h1_v = bufs[2], bufs[3]
            rows0_v, rows1_v = bufs[7], bufs[8]
            cps = []
            for k in range(NB):
                blk = pl.ds(k * BL, BL)
                cps.append(pltpu.async_copy(
                    tab_sh.at[h0_v.at[blk]], rows0_v.at[blk], sem_g))
                cps.append(pltpu.async_copy(
                    tab_sh.at[h1_v.at[blk]], rows1_v.at[blk], sem_g))
            return cps

        def compute(bufs):
            idx0_v, idx1_v = bufs[0], bufs[1]
            sx_v, sy_v, sz_v = bufs[4], bufs[5], bufs[6]
            rows0_v, rows1_v, dx_v, dy_v, dist_v = bufs[7:]

            def row_body(u, carry2):
                lanes = lax.iota(jnp.int32, L)
                for v in range(UNR):
                    rb = L * (UNR * u + v)
                    o = pl.ds(rb, L)
                    r = rb + lanes
                    p0 = (idx0_v[o] & 1) << 2
                    p1 = (idx1_v[o] & 1) << 2
                    dx = (plsc.load_gather(rows0_v, [r, p0])
                          - plsc.load_gather(rows1_v, [r, p1])
                          + sx_v[o])
                    dy = (plsc.load_gather(rows0_v, [r, p0 + 1])
                          - plsc.load_gather(rows1_v, [r, p1 + 1])
                          + sy_v[o])
                    dz = (plsc.load_gather(rows0_v, [r, p0 + 2])
                          - plsc.load_gather(rows1_v, [r, p1 + 2])
                          + sz_v[o])
                    dx_v[o] = dx
                    dy_v[o] = dy
                    sz_v[o] = dz
                    ss = dx * dx + dy * dy + dz * dz
                    dist_v[o] = _dist_from_sumsq(ss)
                return carry2

            lax.fori_loop(0, C // L // UNR, row_body, 0)

        def issue_out(i, bufs):
            sz_v = bufs[6]
            dx_v, dy_v, dist_v = bufs[9], bufs[10], bufs[11]
            base = wid * PPW + i * C
            sl = pl.ds(base, C)
            return [
                pltpu.async_copy(dist_v, dist_hbm.at[sl], sem_out),
                pltpu.async_copy(dx_v, dx_hbm.at[sl], sem_out),
                pltpu.async_copy(dy_v, dy_hbm.at[sl], sem_out),
                pltpu.async_copy(sz_v, dz_hbm.at[sl], sem_out),
            ]

        def pair_body(j, carry):
            ia = 2 * j
            ib = 2 * j + 1
            in_a = issue_in(ia, bufs_a)
            in_b = issue_in(ib, bufs_b)
            for cp in in_a:
                cp.wait()
            halve(bufs_a)
            g_a = issue_gathers(bufs_a)
            for cp in in_b:
                cp.wait()
            halve(bufs_b)
            for cp in g_a:
                cp.wait()
            g_b = issue_gathers(bufs_b)
            compute(bufs_a)
            out_a = issue_out(ia, bufs_a)
            for cp in g_b:
                cp.wait()
            compute(bufs_b)
            out_b = issue_out(ib, bufs_b)
            for cp in out_a:
                cp.wait()
            for cp in out_b:
                cp.wait()
            return carry

        lax.fori_loop(0, NCH // 2, pair_body, 0)

    return nbr_kernel


_NBR_KERNEL = _make_kernel()


@jax.jit
def _run(coords, idx0, idx1, sx, sy, sz):
    return _NBR_KERNEL(coords, idx0, idx1, sx, sy, sz)


def kernel(species, coordinates, neighbor_idxs, shift_values, cutoff):
    del species, cutoff
    coords = jnp.pad(coordinates.reshape(-1, 3),
                     ((0, 0), (0, 1))).reshape(NT, 8)
    dist, dx, dy, dz = _run(
        coords,
        neighbor_idxs[0], neighbor_idxs[1],
        shift_values[:, 0], shift_values[:, 1], shift_values[:, 2])
    diff = jnp.stack([dx, dy, dz], axis=1)
    return neighbor_idxs, dist, diff


# speedup vs baseline: 50.1795x; 1.0025x over previous
import functools

import jax
import jax.numpy as jnp
from jax import lax
from jax.experimental import pallas as pl
from jax.experimental.pallas import tpu as pltpu
from jax.experimental.pallas import tpu_sc as plsc

NA = 100000
NT = NA // 2
NP = 3200000
NC = 2
NS = 16
NW = NC * NS
PPW = NP // NW
BL = 80
NB = 25
C = NB * BL
NCH = PPW // C
NPAIR = NCH // 2
L = 16
UNR = 5


def _dist_from_sumsq(ss):
    xi = lax.bitcast_convert_type(ss, jnp.int32)
    yi = jnp.int32(0x5F3759DF) - (xi >> 1)
    y = lax.bitcast_convert_type(yi, jnp.float32)
    for _ in range(3):
        y = y * (1.5 - 0.5 * ss * y * y)
    return jnp.where(ss > 0.0, ss * y, 0.0)


def _make_kernel():
    mesh = plsc.VectorSubcoreMesh(core_axis_name="c", subcore_axis_name="s")

    buf = lambda shape, dt: pltpu.VMEM(shape, dt)

    @functools.partial(
        pl.kernel,
        mesh=mesh,
        compiler_params=pltpu.CompilerParams(
            use_tc_tiling_on_sc=False, needs_layout_passes=False),
        out_type=[
            jax.ShapeDtypeStruct((NP,), jnp.float32),
            jax.ShapeDtypeStruct((NP,), jnp.float32),
            jax.ShapeDtypeStruct((NP,), jnp.float32),
            jax.ShapeDtypeStruct((NP,), jnp.float32),
        ],
        scratch_types=[
            pltpu.VMEM_SHARED((NT, 8), jnp.float32),
            buf((C,), jnp.int32), buf((C,), jnp.int32), buf((C,), jnp.int32),
            buf((C,), jnp.float32), buf((C,), jnp.float32),
            buf((C,), jnp.float32),
            buf((C, 8), jnp.float32), buf((C, 8), jnp.float32),
            buf((C,), jnp.float32), buf((C,), jnp.float32),
            buf((C,), jnp.float32), buf((C,), jnp.float32),
            buf((C,), jnp.int32), buf((C,), jnp.int32), buf((C,), jnp.int32),
            buf((C,), jnp.float32), buf((C,), jnp.float32),
            buf((C,), jnp.float32),
            buf((C, 8), jnp.float32), buf((C, 8), jnp.float32),
            buf((C,), jnp.float32), buf((C,), jnp.float32),
            buf((C,), jnp.float32), buf((C,), jnp.float32),
            pltpu.SemaphoreType.DMA,
            pltpu.SemaphoreType.DMA,
            pltpu.SemaphoreType.DMA,
        ],
    )
    def nbr_kernel(coords_hbm, idx0_hbm, idx1_hbm, sx_hbm, sy_hbm, sz_hbm,
                   dist_hbm, dx_hbm, dy_hbm, dz_hbm,
                   tab_sh,
                   idx0_a, idx1_a, par_a, sx_a, sy_a, sz_a,
                   rows0_a, rows1_a, dx_a, dy_a, dz_a, dist_a,
                   idx0_b, idx1_b, par_b, sx_b, sy_b, sz_b,
                   rows0_b, rows1_b, dx_b, dy_b, dz_b, dist_b,
                   sem_in, sem_g, sem_out):
        cid = lax.axis_index("c")
        sid = lax.axis_index("s")
        wid = cid * NS + sid

        @pl.when(sid == 0)
        def _stage_table():
            pltpu.sync_copy(coords_hbm, tab_sh)

        plsc.subcore_barrier()

        bufs_a = (idx0_a, idx1_a, par_a, sx_a, sy_a, sz_a,
                  rows0_a, rows1_a, dx_a, dy_a, dz_a, dist_a)
        bufs_b = (idx0_b, idx1_b, par_b, sx_b, sy_b, sz_b,
                  rows0_b, rows1_b, dx_b, dy_b, dz_b, dist_b)

        def in_copies(i, bufs):
            idx0_v, idx1_v = bufs[0], bufs[1]
            sx_v, sy_v, sz_v = bufs[3], bufs[4], bufs[5]
            sl = pl.ds(wid * PPW + i * C, C)
            return [
                pltpu.make_async_copy(idx0_hbm.at[sl], idx0_v, sem_in),
                pltpu.make_async_copy(idx1_hbm.at[sl], idx1_v, sem_in),
                pltpu.make_async_copy(sx_hbm.at[sl], sx_v, sem_in),
                pltpu.make_async_copy(sy_hbm.at[sl], sy_v, sem_in),
                pltpu.make_async_copy(sz_hbm.at[sl], sz_v, sem_in),
            ]

        def issue_in(i, bufs):
            for cp in in_copies(i, bufs):
                cp.start()

        def wait_in(i, bufs):
            for cp in in_copies(i, bufs):
                cp.wait()

        def halve(bufs):
            idx0_v, idx1_v, par_v = bufs[:3]

            def hb(u, carry2):
                o = pl.ds(L * u, L)
                i0 = idx0_v[o]
                i1 = idx1_v[o]
                par_v[o] = ((i0 & 1) << 2) | ((i1 & 1) << 6)
                idx0_v[o] = i0 >> 1
                idx1_v[o] = i1 >> 1
                return carry2

            lax.fori_loop(0, C // L, hb, 0)

        def g_copies(bufs, real):
            idx0_v, idx1_v = bufs[0], bufs[1]
            rows0_v, rows1_v = bufs[6], bufs[7]
            cps = []
            for k in range(NB):
                blk = pl.ds(k * BL, BL)
                if real:
                    s0 = tab_sh.at[idx0_v.at[blk]]
                    s1 = tab_sh.at[idx1_v.at[blk]]
                else:
                    s0 = coords_hbm.at[pl.ds(0, BL)]
                    s1 = coords_hbm.at[pl.ds(0, BL)]
                cps.append(pltpu.make_async_copy(s0, rows0_v.at[blk], sem_g))
                cps.append(pltpu.make_async_copy(s1, rows1_v.at[blk], sem_g))
            return cps

        def issue_g(bufs):
            for cp in g_copies(bufs, True):
                cp.start()

        def wait_g(bufs):
            for cp in g_copies(bufs, False):
                cp.wait()

        def compute(bufs):
            par_v = bufs[2]
            sx_v, sy_v, sz_v = bufs[3], bufs[4], bufs[5]
            rows0_v, rows1_v, dx_v, dy_v, dz_v, dist_v = bufs[6:]

            def row_body(u, carry2):
                lanes = lax.iota(jnp.int32, L)
                for v in range(UNR):
                    rb = L * (UNR * u + v)
                    o = pl.ds(rb, L)
                    r = rb + lanes
                    pv = par_v[o]
                    p0 = pv & 4
                    p1 = (pv >> 4) & 4
                    dx = (plsc.load_gather(rows0_v, [r, p0])
                          - plsc.load_gather(rows1_v, [r, p1])
                          + sx_v[o])
                    dy = (plsc.load_gather(rows0_v, [r, p0 + 1])
                          - plsc.load_gather(rows1_v, [r, p1 + 1])
                          + sy_v[o])
                    dz = (plsc.load_gather(rows0_v, [r, p0 + 2])
                          - plsc.load_gather(rows1_v, [r, p1 + 2])
                          + sz_v[o])
                    dx_v[o] = dx
                    dy_v[o] = dy
                    dz_v[o] = dz
                    ss = dx * dx + dy * dy + dz * dz
                    dist_v[o] = _dist_from_sumsq(ss)
                return carry2

            lax.fori_loop(0, C // L // UNR, row_body, 0)

        def out_copies(i, bufs):
            dx_v, dy_v, dz_v, dist_v = bufs[8:]
            sl = pl.ds(wid * PPW + i * C, C)
            return [
                pltpu.make_async_copy(dist_v, dist_hbm.at[sl], sem_out),
                pltpu.make_async_copy(dx_v, dx_hbm.at[sl], sem_out),
                pltpu.make_async_copy(dy_v, dy_hbm.at[sl], sem_out),
                pltpu.make_async_copy(dz_v, dz_hbm.at[sl], sem_out),
            ]

        def issue_out(i, bufs):
            for cp in out_copies(i, bufs):
                cp.start()

        def wait_out(i, bufs):
            for cp in out_copies(i, bufs):
                cp.wait()

        issue_in(0, bufs_a)
        issue_in(1, bufs_b)
        wait_in(0, bufs_a)
        halve(bufs_a)
        issue_g(bufs_a)

        def pair_body(j, carry):
            ia = 2 * j
            ib = 2 * j + 1
            wait_g(bufs_a)
            wait_in(ib, bufs_b)
            halve(bufs_b)
            issue_g(bufs_b)

            @pl.when(j > 0)
            def _drain_out_a():
                wait_out(ia - 2, bufs_a)

            compute(bufs_a)
            issue_out(ia, bufs_a)

            @pl.when(j < NPAIR - 1)
            def _prefetch_in_a():
                issue_in(ia + 2, bufs_a)

            wait_g(bufs_b)

            @pl.when(j > 0)
            def _drain_out_b():
                wait_out(ib - 2, bufs_b)

            compute(bufs_b)
            issue_out(ib, bufs_b)

            @pl.when(j < NPAIR - 1)
            def _prefetch_next():
                wait_in(ia + 2, bufs_a)
                halve(bufs_a)
                issue_g(bufs_a)
                issue_in(ib + 2, bufs_b)

            return carry

        lax.fori_loop(0, NPAIR, pair_body, 0)

        wait_out(NCH - 2, bufs_a)
        wait_out(NCH - 1, bufs_b)

    return nbr_kernel


_NBR_KERNEL = _make_kernel()


@jax.jit
def _run(coords, idx0, idx1, sx, sy, sz):
    return _NBR_KERNEL(coords, idx0, idx1, sx, sy, sz)


def kernel(species, coordinates, neighbor_idxs, shift_values, cutoff):
    del species, cutoff
    coords = jnp.pad(coordinates.reshape(-1, 3),
                     ((0, 0), (0, 1))).reshape(NT, 8)
    dist, dx, dy, dz = _run(
        coords,
        neighbor_idxs[0], neighbor_idxs[1],
        shift_values[:, 0], shift_values[:, 1], shift_values[:, 2])
    diff = jnp.stack([dx, dy, dz], axis=1)
    return neighbor_idxs, dist, diff


# speedup vs baseline: 52.6251x; 1.0487x over previous
import functools

import jax
import jax.numpy as jnp
from jax import lax
from jax.experimental import pallas as pl
from jax.experimental.pallas import tpu as pltpu
from jax.experimental.pallas import tpu_sc as plsc

NA = 100000
NT = NA // 2
NP = 3200000
NC = 2
NS = 16
NW = NC * NS
PPW = NP // NW
BL = 80
NB = 25
C = NB * BL
NCH = PPW // C
NPAIR = NCH // 2
L = 16
UNR = 5


def _dist_from_sumsq(ss):
    xi = lax.bitcast_convert_type(ss, jnp.int32)
    yi = jnp.int32(0x5F3759DF) - (xi >> 1)
    y = lax.bitcast_convert_type(yi, jnp.float32)
    for _ in range(2):
        y = y * (1.5 - 0.5 * ss * y * y)
    return jnp.where(ss > 0.0, ss * y, 0.0)


def _make_kernel():
    mesh = plsc.VectorSubcoreMesh(core_axis_name="c", subcore_axis_name="s")

    buf = lambda shape, dt: pltpu.VMEM(shape, dt)

    @functools.partial(
        pl.kernel,
        mesh=mesh,
        compiler_params=pltpu.CompilerParams(
            use_tc_tiling_on_sc=False, needs_layout_passes=False),
        out_type=[
            jax.ShapeDtypeStruct((NP,), jnp.float32),
            jax.ShapeDtypeStruct((NP,), jnp.float32),
            jax.ShapeDtypeStruct((NP,), jnp.float32),
            jax.ShapeDtypeStruct((NP,), jnp.float32),
        ],
        scratch_types=[
            pltpu.VMEM_SHARED((NT, 8), jnp.float32),
            buf((C,), jnp.int32), buf((C,), jnp.int32), buf((C,), jnp.int32),
            buf((C,), jnp.float32), buf((C,), jnp.float32),
            buf((C,), jnp.float32),
            buf((C, 8), jnp.float32), buf((C, 8), jnp.float32),
            buf((C,), jnp.float32), buf((C,), jnp.float32),
            buf((C,), jnp.float32), buf((C,), jnp.float32),
            buf((C,), jnp.int32), buf((C,), jnp.int32), buf((C,), jnp.int32),
            buf((C,), jnp.float32), buf((C,), jnp.float32),
            buf((C,), jnp.float32),
            buf((C, 8), jnp.float32), buf((C, 8), jnp.float32),
            buf((C,), jnp.float32), buf((C,), jnp.float32),
            buf((C,), jnp.float32), buf((C,), jnp.float32),
            pltpu.SemaphoreType.DMA,
            pltpu.SemaphoreType.DMA,
            pltpu.SemaphoreType.DMA,
        ],
    )
    def nbr_kernel(coords_hbm, idx0_hbm, idx1_hbm, par_hbm,
                   sx_hbm, sy_hbm, sz_hbm,
                   dist_hbm, dx_hbm, dy_hbm, dz_hbm,
                   tab_sh,
                   idx0_a, idx1_a, par_a, sx_a, sy_a, sz_a,
                   rows0_a, rows1_a, dx_a, dy_a, dz_a, dist_a,
                   idx0_b, idx1_b, par_b, sx_b, sy_b, sz_b,
                   rows0_b, rows1_b, dx_b, dy_b, dz_b, dist_b,
                   sem_in, sem_g, sem_out):
        cid = lax.axis_index("c")
        sid = lax.axis_index("s")
        wid = cid * NS + sid

        @pl.when(sid == 0)
        def _stage_table():
            pltpu.sync_copy(coords_hbm, tab_sh)

        plsc.subcore_barrier()

        bufs_a = (idx0_a, idx1_a, par_a, sx_a, sy_a, sz_a,
                  rows0_a, rows1_a, dx_a, dy_a, dz_a, dist_a)
        bufs_b = (idx0_b, idx1_b, par_b, sx_b, sy_b, sz_b,
                  rows0_b, rows1_b, dx_b, dy_b, dz_b, dist_b)

        def in_copies(i, bufs):
            idx0_v, idx1_v, par_v = bufs[0], bufs[1], bufs[2]
            sx_v, sy_v, sz_v = bufs[3], bufs[4], bufs[5]
            sl = pl.ds(wid * PPW + i * C, C)
            return [
                pltpu.make_async_copy(idx0_hbm.at[sl], idx0_v, sem_in),
                pltpu.make_async_copy(idx1_hbm.at[sl], idx1_v, sem_in),
                pltpu.make_async_copy(par_hbm.at[sl], par_v, sem_in),
                pltpu.make_async_copy(sx_hbm.at[sl], sx_v, sem_in),
                pltpu.make_async_copy(sy_hbm.at[sl], sy_v, sem_in),
                pltpu.make_async_copy(sz_hbm.at[sl], sz_v, sem_in),
            ]

        def issue_in(i, bufs):
            for cp in in_copies(i, bufs):
                cp.start()

        def wait_in(i, bufs):
            for cp in in_copies(i, bufs):
                cp.wait()

        def g_copies(bufs, real):
            idx0_v, idx1_v = bufs[0], bufs[1]
            rows0_v, rows1_v = bufs[6], bufs[7]
            cps = []
            for k in range(NB):
                blk = pl.ds(k * BL, BL)
                if real:
                    s0 = tab_sh.at[idx0_v.at[blk]]
                    s1 = tab_sh.at[idx1_v.at[blk]]
                else:
                    s0 = coords_hbm.at[pl.ds(0, BL)]
                    s1 = coords_hbm.at[pl.ds(0, BL)]
                cps.append(pltpu.make_async_copy(s0, rows0_v.at[blk], sem_g))
                cps.append(pltpu.make_async_copy(s1, rows1_v.at[blk], sem_g))
            return cps

        def issue_g(bufs):
            for cp in g_copies(bufs, True):
                cp.start()

        def wait_g(bufs):
            for cp in g_copies(bufs, False):
                cp.wait()

        def compute(bufs):
            par_v = bufs[2]
            sx_v, sy_v, sz_v = bufs[3], bufs[4], bufs[5]
            rows0_v, rows1_v, dx_v, dy_v, dz_v, dist_v = bufs[6:]

            def row_body(u, carry2):
                lanes = lax.iota(jnp.int32, L)
                for v in range(UNR):
                    rb = L * (UNR * u + v)
                    o = pl.ds(rb, L)
                    r = rb + lanes
                    pv = par_v[o]
                    p0 = pv & 4
                    p1 = (pv >> 4) & 4
                    dx = (plsc.load_gather(rows0_v, [r, p0])
                          - plsc.load_gather(rows1_v, [r, p1])
                          + sx_v[o])
                    dy = (plsc.load_gather(rows0_v, [r, p0 + 1])
                          - plsc.load_gather(rows1_v, [r, p1 + 1])
                          + sy_v[o])
                    dz = (plsc.load_gather(rows0_v, [r, p0 + 2])
                          - plsc.load_gather(rows1_v, [r, p1 + 2])
                          + sz_v[o])
                    dx_v[o] = dx
                    dy_v[o] = dy
                    dz_v[o] = dz
                    ss = dx * dx + dy * dy + dz * dz
                    dist_v[o] = _dist_from_sumsq(ss)
                return carry2

            lax.fori_loop(0, C // L // UNR, row_body, 0)

        def out_copies(i, bufs):
            dx_v, dy_v, dz_v, dist_v = bufs[8:]
            sl = pl.ds(wid * PPW + i * C, C)
            return [
                pltpu.make_async_copy(dist_v, dist_hbm.at[sl], sem_out),
                pltpu.make_async_copy(dx_v, dx_hbm.at[sl], sem_out),
                pltpu.make_async_copy(dy_v, dy_hbm.at[sl], sem_out),
                pltpu.make_async_copy(dz_v, dz_hbm.at[sl], sem_out),
            ]

        def issue_out(i, bufs):
            for cp in out_copies(i, bufs):
                cp.start()

        def wait_out(i, bufs):
            for cp in out_copies(i, bufs):
                cp.wait()

        issue_in(0, bufs_a)
        issue_in(1, bufs_b)
        wait_in(0, bufs_a)
        issue_g(bufs_a)

        def pair_body(j, carry):
            ia = 2 * j
            ib = 2 * j + 1
            wait_g(bufs_a)
            wait_in(ib, bufs_b)
            issue_g(bufs_b)

            @pl.when(j > 0)
            def _drain_out_a():
                wait_out(ia - 2, bufs_a)

            compute(bufs_a)
            issue_out(ia, bufs_a)

            @pl.when(j < NPAIR - 1)
            def _prefetch_in_a():
                issue_in(ia + 2, bufs_a)

            wait_g(bufs_b)

            @pl.when(j > 0)
            def _drain_out_b():
                wait_out(ib - 2, bufs_b)

            compute(bufs_b)
            issue_out(ib, bufs_b)

            @pl.when(j < NPAIR - 1)
            def _prefetch_next():
                wait_in(ia + 2, bufs_a)
                issue_g(bufs_a)
                issue_in(ib + 2, bufs_b)

            return carry

        lax.fori_loop(0, NPAIR, pair_body, 0)

        wait_out(NCH - 2, bufs_a)
        wait_out(NCH - 1, bufs_b)

    return nbr_kernel


_NBR_KERNEL = _make_kernel()


@jax.jit
def _run(coords, h0, h1, par, sx, sy, sz):
    return _NBR_KERNEL(coords, h0, h1, par, sx, sy, sz)


def kernel(species, coordinates, neighbor_idxs, shift_values, cutoff):
    del species, cutoff
    coords = jnp.pad(coordinates.reshape(-1, 3),
                     ((0, 0), (0, 1))).reshape(NT, 8)
    i0 = neighbor_idxs[0]
    i1 = neighbor_idxs[1]
    par = ((i0 & 1) << 2) | ((i1 & 1) << 6)
    dist, dx, dy, dz = _run(
        coords, i0 >> 1, i1 >> 1, par,
        shift_values[:, 0], shift_values[:, 1], shift_values[:, 2])
    diff = jnp.stack([dx, dy, dz], axis=1)
    return neighbor_idxs, dist, diff
